# baseline (speedup 1.0000x reference)
import functools

import jax
import jax.numpy as jnp
from jax.experimental import pallas as pl
from jax.experimental.pallas import tpu as pltpu

DIM = 4096
NUM_EXPERTS = 64
BLOCK_T = 1024
KSPLIT = 2
BLOCK_K = DIM // KSPLIT


def _router_body(x_ref, w_ref, topi_ref, wts_ref, pri_ref, acc_ref, *, block_t):
    k = pl.program_id(1)
    partial = jax.lax.dot_general(
        x_ref[...], w_ref[:, pl.ds(k * BLOCK_K, BLOCK_K)],
        dimension_numbers=(((1,), (1,)), ((), ())),
        preferred_element_type=jnp.float32,
    )

    @pl.when(k == 0)
    def _():
        acc_ref[...] = partial

    @pl.when(k > 0)
    def _():
        acc_ref[...] += partial

    @pl.when(k == KSPLIT - 1)
    def _():
        logits = acc_ref[...]
        m = jnp.max(logits, axis=1, keepdims=True)
        idx = jnp.argmax(logits, axis=1)
        s = jnp.sum(jnp.exp(logits - m), axis=1)
        topi_ref[...] = idx.astype(jnp.int32).reshape(1, 1, block_t)
        wts_ref[...] = (1.0 / s).reshape(1, 1, block_t)
        row0 = pl.program_id(0) * block_t
        rows = row0 + jax.lax.broadcasted_iota(jnp.int32, (1, 1, block_t), 2)
        pri_ref[...] = -rows.astype(jnp.float32)


@jax.jit
def kernel(x, gate_w):
    t = x.shape[0]
    nb = t // BLOCK_T
    grid = (nb, KSPLIT)
    topi, wts, pri = pl.pallas_call(
        functools.partial(_router_body, block_t=BLOCK_T),
        grid=grid,
        in_specs=[
            pl.BlockSpec((BLOCK_T, BLOCK_K), lambda i, k: (i, k)),
            pl.BlockSpec((NUM_EXPERTS, DIM), lambda i, k: (0, 0)),
        ],
        out_specs=[
            pl.BlockSpec((1, 1, BLOCK_T), lambda i, k: (i, 0, 0)),
            pl.BlockSpec((1, 1, BLOCK_T), lambda i, k: (i, 0, 0)),
            pl.BlockSpec((1, 1, BLOCK_T), lambda i, k: (i, 0, 0)),
        ],
        out_shape=[
            jax.ShapeDtypeStruct((nb, 1, BLOCK_T), jnp.int32),
            jax.ShapeDtypeStruct((nb, 1, BLOCK_T), jnp.float32),
            jax.ShapeDtypeStruct((nb, 1, BLOCK_T), jnp.float32),
        ],
        scratch_shapes=[pltpu.VMEM((BLOCK_T, NUM_EXPERTS), jnp.float32)],
        compiler_params=pltpu.CompilerParams(
            vmem_limit_bytes=128 * 1024 * 1024),
    )(x, gate_w)
    return (topi.reshape(t, 1), wts.reshape(t, 1), pri.reshape(t))
